# baseline (speedup 1.0000x reference)
import functools

import jax
import jax.numpy as jnp
from jax import lax
from jax.experimental import pallas as pl
from jax.experimental.pallas import tpu as pltpu
from jax.experimental.pallas import tpu_sc as plsc

_NUM_SC_CORES = 2
_NUM_SC_SUBCORES = 16
_GATHER_CHUNK = 32
_GROUP_SIZES = (1, 7)
_BJ = 1024


def _sc_gather(table, idx):
    n_rows, row_dim = idx.shape[0], table.shape[1]
    n_workers = _NUM_SC_CORES * _NUM_SC_SUBCORES
    per_worker = n_rows // n_workers
    chunk = min(_GATHER_CHUNK, per_worker)
    n_chunks = per_worker // chunk

    mesh = plsc.VectorSubcoreMesh(core_axis_name="c", subcore_axis_name="s")

    @functools.partial(
        pl.kernel,
        mesh=mesh,
        out_type=jax.ShapeDtypeStruct((n_rows, row_dim), table.dtype),
        scratch_types=[
            pltpu.VMEM((per_worker,), jnp.int32),
            pltpu.VMEM((chunk, row_dim), table.dtype),
            pltpu.SemaphoreType.DMA,
        ],
    )
    def gather_kernel(table_hbm, idx_hbm, out_hbm, idx_v, rows_v, sem):
        wid = lax.axis_index("s") * _NUM_SC_CORES + lax.axis_index("c")
        base = wid * per_worker
        pltpu.sync_copy(idx_hbm.at[pl.ds(base, per_worker)], idx_v)

        @pl.loop(0, n_chunks)
        def _(c):
            off = c * chunk
            pltpu.async_copy(
                table_hbm.at[idx_v.at[pl.ds(off, chunk)]], rows_v, sem
            ).wait()
            pltpu.sync_copy(rows_v, out_hbm.at[pl.ds(base + off, chunk)])

    return gather_kernel(table, idx)


def _pack_x(x_flat):
    R, I = x_flat.shape
    BR = 512

    def pack_body(x_ref, o_ref):
        xa = x_ref[:, : I // 2].astype(jnp.bfloat16).astype(jnp.float32)
        xb = x_ref[:, I // 2 :].astype(jnp.bfloat16).astype(jnp.float32)
        a = lax.bitcast_convert_type(xa, jnp.uint32)
        b = lax.bitcast_convert_type(xb, jnp.uint32)
        o_ref[...] = (a | (b >> 16)).astype(jnp.int32)

    return pl.pallas_call(
        pack_body,
        grid=(R // BR,),
        in_specs=[pl.BlockSpec((BR, I), lambda r: (r, 0))],
        out_specs=pl.BlockSpec((BR, I // 2), lambda r: (r, 0)),
        out_shape=jax.ShapeDtypeStruct((R, I // 2), jnp.int32),
    )(x_flat)


def _mm_group(y, x_g, W, e0, n_e, out_shape):
    B, E, K, J = out_shape
    I = W.shape[1]
    JB = J // _BJ

    def mm_body(*refs):
        x_ref, w_ref, o_ref, xbf_ref = refs[-4:]

        @pl.when(pl.program_id(1) == 0)
        def _():
            u = lax.bitcast_convert_type(x_ref[0], jnp.uint32)
            hi = lax.bitcast_convert_type(u & jnp.uint32(0xFFFF0000), jnp.float32)
            lo = lax.bitcast_convert_type(u << 16, jnp.float32)
            xbf_ref[:, : I // 2] = hi.astype(jnp.bfloat16)
            xbf_ref[:, I // 2 :] = lo.astype(jnp.bfloat16)

        o_ref[0, 0] = lax.dot_general(
            xbf_ref[...],
            w_ref[0].astype(jnp.bfloat16),
            (((1,), (0,)), ((), ())),
            preferred_element_type=jnp.float32,
        )

    y_args = () if y is None else (y,)
    y_specs = [] if y is None else [pl.BlockSpec(memory_space=pl.MemorySpace.ANY)]
    aliases = {} if y is None else {0: 0}
    return pl.pallas_call(
        mm_body,
        grid=(n_e, JB),
        in_specs=y_specs
        + [
            pl.BlockSpec((1, K, I // 2), lambda e, j: (e, 0, 0)),
            pl.BlockSpec((1, I, _BJ), lambda e, j: (e0 + e, 0, j)),
        ],
        out_specs=pl.BlockSpec((1, 1, K, _BJ), lambda e, j: (0, e0 + e, 0, j)),
        out_shape=jax.ShapeDtypeStruct((B, E, K, J), jnp.float32),
        scratch_shapes=[pltpu.VMEM((K, I), jnp.bfloat16)],
        input_output_aliases=aliases,
    )(*y_args, x_g, W)


def kernel(X, ind, W):
    B, T, I = X.shape
    E, _, J = W.shape
    K = ind.shape[2]
    groups = []
    e0 = 0
    while e0 < B * E:
        for n_e in _GROUP_SIZES:
            if e0 < B * E:
                groups.append((e0, min(n_e, B * E - e0)))
                e0 += n_e

    table = _pack_x(X.reshape(B * T, I))
    offset = (jnp.arange(B, dtype=jnp.int32) * T).reshape(B, 1, 1)
    idx = (ind.astype(jnp.int32) + offset).reshape(B * E * K)

    gathered = [
        _sc_gather(table, idx[e0 * K : (e0 + n_e) * K]).reshape(n_e, K, I // 2)
        for e0, n_e in groups
    ]

    y = None
    for (e0, n_e), x_g in zip(groups, gathered):
        y = _mm_group(y, x_g, W, e0 % E, n_e, (B, E, K, J))
    return y


# speedup vs baseline: 1.0083x; 1.0083x over previous
import functools

import jax
import jax.numpy as jnp
from jax import lax
from jax.experimental import pallas as pl
from jax.experimental.pallas import tpu as pltpu
from jax.experimental.pallas import tpu_sc as plsc

_NUM_SC_CORES = 2
_NUM_SC_SUBCORES = 16
_GATHER_CHUNK = 32
_GROUP_SIZES = (2, 6)
_BJ = 1024


def _sc_gather(table, idx):
    n_rows, row_dim = idx.shape[0], table.shape[1]
    n_workers = _NUM_SC_CORES * _NUM_SC_SUBCORES
    per_worker = n_rows // n_workers
    chunk = min(_GATHER_CHUNK, per_worker)
    n_chunks = per_worker // chunk

    mesh = plsc.VectorSubcoreMesh(core_axis_name="c", subcore_axis_name="s")

    @functools.partial(
        pl.kernel,
        mesh=mesh,
        out_type=jax.ShapeDtypeStruct((n_rows, row_dim), table.dtype),
        scratch_types=[
            pltpu.VMEM((per_worker,), jnp.int32),
            pltpu.VMEM((chunk, row_dim), table.dtype),
            pltpu.VMEM((chunk, row_dim), table.dtype),
            pltpu.SemaphoreType.DMA,
            pltpu.SemaphoreType.DMA,
        ],
    )
    def gather_kernel(table_hbm, idx_hbm, out_hbm, idx_v, rows_a, rows_b, sem_a, sem_b):
        wid = lax.axis_index("s") * _NUM_SC_CORES + lax.axis_index("c")
        base = wid * per_worker
        pltpu.sync_copy(idx_hbm.at[pl.ds(base, per_worker)], idx_v)

        bufs = (rows_a, rows_b)
        sems = (sem_a, sem_b)
        pending = [None, None]
        for c in range(n_chunks):
            b = c % 2
            pending[b] = pltpu.async_copy(
                table_hbm.at[idx_v.at[pl.ds(c * chunk, chunk)]], bufs[b], sems[b]
            )
            if c >= 1:
                p = (c - 1) % 2
                pending[p].wait()
                pltpu.sync_copy(
                    bufs[p], out_hbm.at[pl.ds(base + (c - 1) * chunk, chunk)]
                )
        last = (n_chunks - 1) % 2
        pending[last].wait()
        pltpu.sync_copy(
            bufs[last], out_hbm.at[pl.ds(base + (n_chunks - 1) * chunk, chunk)]
        )

    return gather_kernel(table, idx)


def _pack_x(x_flat):
    R, I = x_flat.shape
    BR = 512

    def pack_body(x_ref, o_ref):
        xa = x_ref[:, : I // 2].astype(jnp.bfloat16).astype(jnp.float32)
        xb = x_ref[:, I // 2 :].astype(jnp.bfloat16).astype(jnp.float32)
        a = lax.bitcast_convert_type(xa, jnp.uint32)
        b = lax.bitcast_convert_type(xb, jnp.uint32)
        o_ref[...] = (a | (b >> 16)).astype(jnp.int32)

    return pl.pallas_call(
        pack_body,
        grid=(R // BR,),
        in_specs=[pl.BlockSpec((BR, I), lambda r: (r, 0))],
        out_specs=pl.BlockSpec((BR, I // 2), lambda r: (r, 0)),
        out_shape=jax.ShapeDtypeStruct((R, I // 2), jnp.int32),
    )(x_flat)


def _mm_group(y, x_g, W, e0, n_e, out_shape):
    B, E, K, J = out_shape
    I = W.shape[1]
    JB = J // _BJ

    def mm_body(*refs):
        x_ref, w_ref, o_ref, xbf_ref = refs[-4:]

        @pl.when(pl.program_id(1) == 0)
        def _():
            u = lax.bitcast_convert_type(x_ref[0], jnp.uint32)
            hi = lax.bitcast_convert_type(u & jnp.uint32(0xFFFF0000), jnp.float32)
            lo = lax.bitcast_convert_type(u << 16, jnp.float32)
            xbf_ref[:, : I // 2] = hi.astype(jnp.bfloat16)
            xbf_ref[:, I // 2 :] = lo.astype(jnp.bfloat16)

        o_ref[0, 0] = lax.dot_general(
            xbf_ref[...],
            w_ref[0].astype(jnp.bfloat16),
            (((1,), (0,)), ((), ())),
            preferred_element_type=jnp.float32,
        )

    y_args = () if y is None else (y,)
    y_specs = [] if y is None else [pl.BlockSpec(memory_space=pl.MemorySpace.ANY)]
    aliases = {} if y is None else {0: 0}
    return pl.pallas_call(
        mm_body,
        grid=(n_e, JB),
        in_specs=y_specs
        + [
            pl.BlockSpec((1, K, I // 2), lambda e, j: (e, 0, 0)),
            pl.BlockSpec((1, I, _BJ), lambda e, j: (e0 + e, 0, j)),
        ],
        out_specs=pl.BlockSpec((1, 1, K, _BJ), lambda e, j: (0, e0 + e, 0, j)),
        out_shape=jax.ShapeDtypeStruct((B, E, K, J), jnp.float32),
        scratch_shapes=[pltpu.VMEM((K, I), jnp.bfloat16)],
        input_output_aliases=aliases,
    )(*y_args, x_g, W)


def kernel(X, ind, W):
    B, T, I = X.shape
    E, _, J = W.shape
    K = ind.shape[2]
    groups = []
    e0 = 0
    while e0 < B * E:
        for n_e in _GROUP_SIZES:
            if e0 < B * E:
                groups.append((e0, min(n_e, B * E - e0)))
                e0 += n_e

    table = _pack_x(X.reshape(B * T, I))
    offset = (jnp.arange(B, dtype=jnp.int32) * T).reshape(B, 1, 1)
    idx = (ind.astype(jnp.int32) + offset).reshape(B * E * K)

    gathered = [
        _sc_gather(table, idx[e0 * K : (e0 + n_e) * K]).reshape(n_e, K, I // 2)
        for e0, n_e in groups
    ]

    y = None
    for (e0, n_e), x_g in zip(groups, gathered):
        y = _mm_group(y, x_g, W, e0 % E, n_e, (B, E, K, J))
    return y


# speedup vs baseline: 1.0254x; 1.0169x over previous
import functools

import jax
import jax.numpy as jnp
from jax import lax
from jax.experimental import pallas as pl
from jax.experimental.pallas import tpu as pltpu
from jax.experimental.pallas import tpu_sc as plsc

_NUM_SC_CORES = 2
_NUM_SC_SUBCORES = 16
_GATHER_CHUNK = 32
_GROUP_SIZES = (2, 6)
_BJ = 1024


def _sc_gather(table, idx):
    n_rows, row_dim = idx.shape[0], table.shape[1]
    n_workers = _NUM_SC_CORES * _NUM_SC_SUBCORES
    per_worker = n_rows // n_workers
    chunk = min(_GATHER_CHUNK, per_worker)
    n_chunks = per_worker // chunk

    mesh = plsc.VectorSubcoreMesh(core_axis_name="c", subcore_axis_name="s")

    @functools.partial(
        pl.kernel,
        mesh=mesh,
        out_type=jax.ShapeDtypeStruct((n_rows, row_dim), table.dtype),
        scratch_types=[
            pltpu.VMEM((per_worker,), jnp.int32),
            pltpu.VMEM((chunk, row_dim), table.dtype),
            pltpu.SemaphoreType.DMA,
        ],
    )
    def gather_kernel(table_hbm, idx_hbm, out_hbm, idx_v, rows_v, sem):
        wid = lax.axis_index("s") * _NUM_SC_CORES + lax.axis_index("c")
        base = wid * per_worker
        pltpu.sync_copy(idx_hbm.at[pl.ds(base, per_worker)], idx_v)

        @pl.loop(0, n_chunks)
        def _(c):
            off = c * chunk
            pltpu.async_copy(
                table_hbm.at[idx_v.at[pl.ds(off, chunk)]], rows_v, sem
            ).wait()
            pltpu.sync_copy(rows_v, out_hbm.at[pl.ds(base + off, chunk)])

    return gather_kernel(table, idx)


def _pack_x(x_flat):
    R, I = x_flat.shape
    BR = 512

    def pack_body(x_ref, o_ref):
        xa = x_ref[:, : I // 2].astype(jnp.bfloat16).astype(jnp.float32)
        xb = x_ref[:, I // 2 :].astype(jnp.bfloat16).astype(jnp.float32)
        a = lax.bitcast_convert_type(xa, jnp.uint32)
        b = lax.bitcast_convert_type(xb, jnp.uint32)
        o_ref[...] = (a | (b >> 16)).astype(jnp.int32)

    return pl.pallas_call(
        pack_body,
        grid=(R // BR,),
        in_specs=[pl.BlockSpec((BR, I), lambda r: (r, 0))],
        out_specs=pl.BlockSpec((BR, I // 2), lambda r: (r, 0)),
        out_shape=jax.ShapeDtypeStruct((R, I // 2), jnp.int32),
    )(x_flat)


def _mm_group(y, x_g, W, e0, n_e, out_shape):
    B, E, K, J = out_shape
    I = W.shape[1]
    JB = J // _BJ

    def mm_body(*refs):
        x_ref, w_ref, o_ref, xbf_ref = refs[-4:]

        @pl.when(pl.program_id(1) == 0)
        def _():
            u = lax.bitcast_convert_type(x_ref[0], jnp.uint32)
            hi = lax.bitcast_convert_type(u & jnp.uint32(0xFFFF0000), jnp.float32)
            lo = lax.bitcast_convert_type(u << 16, jnp.float32)
            xbf_ref[:, : I // 2] = hi.astype(jnp.bfloat16)
            xbf_ref[:, I // 2 :] = lo.astype(jnp.bfloat16)

        o_ref[0, 0] = lax.dot_general(
            xbf_ref[...],
            w_ref[0].astype(jnp.bfloat16),
            (((1,), (0,)), ((), ())),
            preferred_element_type=jnp.float32,
        )

    y_args = () if y is None else (y,)
    y_specs = [] if y is None else [pl.BlockSpec(memory_space=pl.MemorySpace.ANY)]
    aliases = {} if y is None else {0: 0}
    return pl.pallas_call(
        mm_body,
        grid=(n_e, JB),
        in_specs=y_specs
        + [
            pl.BlockSpec((1, K, I // 2), lambda e, j: (e, 0, 0)),
            pl.BlockSpec((1, I, _BJ), lambda e, j: (e0 + e, 0, j)),
        ],
        out_specs=pl.BlockSpec((1, 1, K, _BJ), lambda e, j: (0, e0 + e, 0, j)),
        out_shape=jax.ShapeDtypeStruct((B, E, K, J), jnp.float32),
        scratch_shapes=[pltpu.VMEM((K, I), jnp.bfloat16)],
        input_output_aliases=aliases,
    )(*y_args, x_g, W)


def kernel(X, ind, W):
    B, T, I = X.shape
    E, _, J = W.shape
    K = ind.shape[2]
    groups = []
    e0 = 0
    while e0 < B * E:
        for n_e in _GROUP_SIZES:
            if e0 < B * E:
                groups.append((e0, min(n_e, B * E - e0)))
                e0 += n_e

    table = _pack_x(X.reshape(B * T, I))
    offset = (jnp.arange(B, dtype=jnp.int32) * T).reshape(B, 1, 1)
    idx = (ind.astype(jnp.int32) + offset).reshape(B * E * K)

    gathered = [
        _sc_gather(table, idx[e0 * K : (e0 + n_e) * K]).reshape(n_e, K, I // 2)
        for e0, n_e in groups
    ]

    y = None
    for (e0, n_e), x_g in zip(groups, gathered):
        y = _mm_group(y, x_g, W, e0 % E, n_e, (B, E, K, J))
    return y
